# Initial kernel scaffold; baseline (speedup 1.0000x reference)
import jax
import jax.numpy as jnp
from jax.experimental import pallas as pl


def kernel(input_ids, token_table, pos_table):
    raise NotImplementedError("write your pallas kernel here")


# speedup vs baseline: 1.1473x; 1.1473x over previous
import functools

import jax
import jax.numpy as jnp
from jax import lax
from jax.experimental import pallas as pl
from jax.experimental.pallas import tpu as pltpu
from jax.experimental.pallas import tpu_sc as plsc

_LANES = 16


@functools.lru_cache(maxsize=None)
def _build(B, S, H, NC, NS):
    NW = NC * NS
    s_per_w = S // NW

    mesh = plsc.VectorSubcoreMesh(core_axis_name="c", subcore_axis_name="s")

    @functools.partial(
        pl.kernel,
        out_type=jax.ShapeDtypeStruct((B * S, H), jnp.float32),
        mesh=mesh,
        scratch_types=[
            pltpu.VMEM((s_per_w,), jnp.int32),
            pltpu.VMEM((s_per_w, H), jnp.float32),
            pltpu.VMEM((s_per_w, H), jnp.float32),
            pltpu.SemaphoreType.DMA,
        ],
    )
    def emb(ids_hbm, tok_hbm, pos_hbm, out_hbm, idx_v, tok_v, pos_v, sem):
        wid = lax.axis_index("s") * NC + lax.axis_index("c")
        s_base = wid * s_per_w
        pltpu.sync_copy(pos_hbm.at[pl.ds(s_base, s_per_w)], pos_v)
        for b in range(B):
            flat_base = b * S + s_base
            pltpu.sync_copy(ids_hbm.at[pl.ds(flat_base, s_per_w)], idx_v)
            pltpu.async_copy(tok_hbm.at[idx_v], tok_v, sem).wait()

            def row_add(r, _):
                for j in range(H // _LANES):
                    sl = pl.ds(j * _LANES, _LANES)
                    tok_v[r, sl] = tok_v[r, sl] + pos_v[r, sl]
                return 0

            lax.fori_loop(0, s_per_w, row_add, 0)
            pltpu.sync_copy(tok_v, out_hbm.at[pl.ds(flat_base, s_per_w)])

    return emb


def kernel(input_ids, token_table, pos_table):
    B, S = input_ids.shape
    H = token_table.shape[1]
    info = plsc.get_sparse_core_info()
    emb = _build(B, S, H, info.num_cores, info.num_subcores)
    ids_flat = input_ids.reshape(-1).astype(jnp.int32)
    out = emb(ids_flat, token_table, pos_table)
    return out.reshape(B, S, H)


# speedup vs baseline: 1.3926x; 1.2138x over previous
import functools

import jax
import jax.numpy as jnp
from jax import lax
from jax.experimental import pallas as pl
from jax.experimental.pallas import tpu as pltpu
from jax.experimental.pallas import tpu_sc as plsc

_LANES = 16


@functools.lru_cache(maxsize=None)
def _build(B, S, H, NC, NS):
    NW = NC * NS
    s_per_w = S // NW

    mesh = plsc.VectorSubcoreMesh(core_axis_name="c", subcore_axis_name="s")

    @functools.partial(
        pl.kernel,
        out_type=jax.ShapeDtypeStruct((B * S, H), jnp.float32),
        mesh=mesh,
        scratch_types=[
            *[pltpu.VMEM((s_per_w,), jnp.int32) for _ in range(B)],
            pltpu.VMEM((s_per_w, H), jnp.float32),
            pltpu.VMEM((s_per_w, H), jnp.float32),
            pltpu.VMEM((s_per_w, H), jnp.float32),
            pltpu.SemaphoreType.DMA,
            pltpu.SemaphoreType.DMA,
            pltpu.SemaphoreType.DMA,
            pltpu.SemaphoreType.DMA,
            pltpu.SemaphoreType.DMA,
        ],
    )
    def emb(ids_hbm, tok_hbm, pos_hbm, out_hbm,
            *refs):
        idx = refs[:B]
        tok0, tok1, pos_v, g0, g1, s0, s1, psem = refs[B:]
        wid = lax.axis_index("s") * NC + lax.axis_index("c")
        s_base = wid * s_per_w
        tok = (tok0, tok1)
        gsem = (g0, g1)
        ssem = (s0, s1)

        idx_cps = [
            pltpu.async_copy(ids_hbm.at[b, pl.ds(s_base, s_per_w)], idx[b], psem)
            for b in range(B)
        ]
        for cp in idx_cps:
            cp.wait()
        gathers = [None] * B
        stores = [None] * B
        gathers[0] = pltpu.async_copy(tok_hbm.at[idx[0]], tok[0], gsem[0])
        pos_cp = pltpu.async_copy(pos_hbm.at[pl.ds(s_base, s_per_w)], pos_v, psem)

        def row_add(t_ref, r, _):
            for j in range(H // _LANES):
                sl = pl.ds(j * _LANES, _LANES)
                t_ref[r, sl] = t_ref[r, sl] + pos_v[r, sl]
            return 0

        for b in range(B):
            buf = b % 2
            if b + 1 < B:
                nbuf = (b + 1) % 2
                if b >= 1:
                    stores[b - 1].wait()
                gathers[b + 1] = pltpu.async_copy(
                    tok_hbm.at[idx[b + 1]], tok[nbuf], gsem[nbuf])
            if b == 0:
                pos_cp.wait()
            gathers[b].wait()
            lax.fori_loop(0, s_per_w, functools.partial(row_add, tok[buf]), 0)
            stores[b] = pltpu.async_copy(
                tok[buf], out_hbm.at[pl.ds(b * S + s_base, s_per_w)], ssem[buf])
        for st in stores[max(0, B - 2):]:
            st.wait()

    return emb


def kernel(input_ids, token_table, pos_table):
    B, S = input_ids.shape
    H = token_table.shape[1]
    info = plsc.get_sparse_core_info()
    emb = _build(B, S, H, info.num_cores, info.num_subcores)
    ids = input_ids.astype(jnp.int32)
    out = emb(ids, token_table, pos_table)
    return out.reshape(B, S, H)


# speedup vs baseline: 1.4021x; 1.0068x over previous
import functools

import jax
import jax.numpy as jnp
from jax import lax
from jax.experimental import pallas as pl
from jax.experimental.pallas import tpu as pltpu
from jax.experimental.pallas import tpu_sc as plsc

_LANES = 16


@functools.lru_cache(maxsize=None)
def _build(B, S, H, NC, NS):
    NW = NC * NS
    s_per_w = S // NW

    mesh = plsc.VectorSubcoreMesh(core_axis_name="c", subcore_axis_name="s")

    @functools.partial(
        pl.kernel,
        out_type=jax.ShapeDtypeStruct((B * S, H), jnp.float32),
        mesh=mesh,
        scratch_types=[
            *[pltpu.VMEM((s_per_w,), jnp.int32) for _ in range(B)],
            pltpu.VMEM((s_per_w, H), jnp.float32),
            pltpu.VMEM((s_per_w, H), jnp.float32),
            pltpu.VMEM((s_per_w, H), jnp.float32),
            pltpu.SemaphoreType.DMA,
            pltpu.SemaphoreType.DMA,
            pltpu.SemaphoreType.DMA,
            pltpu.SemaphoreType.DMA,
            pltpu.SemaphoreType.DMA,
        ],
    )
    def emb(ids_hbm, tok_hbm, pos_hbm, out_hbm,
            *refs):
        idx = refs[:B]
        tok0, tok1, pos_v, g0, g1, s0, s1, psem = refs[B:]
        wid = lax.axis_index("s") * NC + lax.axis_index("c")
        s_base = wid * s_per_w
        tok = (tok0, tok1)
        gsem = (g0, g1)
        ssem = (s0, s1)

        idx_cps = [
            pltpu.async_copy(ids_hbm.at[b, pl.ds(s_base, s_per_w)], idx[b], psem)
            for b in range(B)
        ]
        for cp in idx_cps:
            cp.wait()
        gathers = [None] * B
        stores = [None] * B
        gathers[0] = pltpu.async_copy(tok_hbm.at[idx[0]], tok[0], gsem[0])
        pos_cp = pltpu.async_copy(pos_hbm.at[pl.ds(s_base, s_per_w)], pos_v, psem)

        def row_add(t_ref, r, _):
            for j in range(H // _LANES):
                sl = pl.ds(j * _LANES, _LANES)
                plsc.addupdate(t_ref.at[r, sl], pos_v[r, sl])
            return 0

        for b in range(B):
            buf = b % 2
            if b + 1 < B:
                nbuf = (b + 1) % 2
                if b >= 1:
                    stores[b - 1].wait()
                gathers[b + 1] = pltpu.async_copy(
                    tok_hbm.at[idx[b + 1]], tok[nbuf], gsem[nbuf])
            if b == 0:
                pos_cp.wait()
            gathers[b].wait()
            lax.fori_loop(0, s_per_w, functools.partial(row_add, tok[buf]), 0)
            stores[b] = pltpu.async_copy(
                tok[buf], out_hbm.at[pl.ds(b * S + s_base, s_per_w)], ssem[buf])
        for st in stores[max(0, B - 2):]:
            st.wait()

    return emb


def kernel(input_ids, token_table, pos_table):
    B, S = input_ids.shape
    H = token_table.shape[1]
    info = plsc.get_sparse_core_info()
    emb = _build(B, S, H, info.num_cores, info.num_subcores)
    ids = input_ids.astype(jnp.int32)
    out = emb(ids, token_table, pos_table)
    return out.reshape(B, S, H)
